# Initial kernel scaffold; baseline (speedup 1.0000x reference)
import jax
import jax.numpy as jnp
from jax.experimental import pallas as pl


def kernel(z, weight):
    raise NotImplementedError("write your pallas kernel here")


# speedup vs baseline: 178.6032x; 178.6032x over previous
import functools

import jax
import jax.numpy as jnp
from jax import lax
from jax.experimental import pallas as pl
from jax.experimental.pallas import tpu as pltpu
from jax.experimental.pallas import tpu_sc as plsc

_NUM_EMBED = 8192
_EMBED_DIM = 256
_BETA = 0.25
_TOKENS = 16384

_TM = 256
_NW = 32
_TOK_PER_W = _TOKENS // _NW
_CHUNK = 128
_NCHUNK = _TOK_PER_W // _CHUNK
_EPI_TM = 2048


def _normalize_body(w_ref, out_ref):
    w = w_ref[...]
    n = jnp.sqrt(jnp.sum(w * w, axis=1, keepdims=True))
    out_ref[...] = w / jnp.maximum(n, 1e-12)


def _argmax_body(z_ref, nw_ref, idx_ref):
    zt = z_ref[...]
    n = jnp.sqrt(jnp.sum(zt * zt, axis=1, keepdims=True))
    zn = zt / jnp.maximum(n, 1e-12)
    scores = lax.dot_general(
        zn, nw_ref[...], (((1,), (1,)), ((), ())),
        preferred_element_type=jnp.float32)
    m = jnp.max(scores, axis=1, keepdims=True)
    ii = lax.broadcasted_iota(jnp.int32, scores.shape, 1)
    idx_ref[0, 0, :] = jnp.max(jnp.where(scores == m, ii, -1), axis=1)


def _sc_body(w_hbm, idx_hbm, zq_hbm, cnt_hbm,
             idx_v, rows_v, ones_v, zeros_v, cnt_sh, sem):
    cid = lax.axis_index("c")
    sid = lax.axis_index("s")
    wid = cid * 16 + sid

    pltpu.sync_copy(idx_hbm.at[pl.ds(wid * _NCHUNK, _NCHUNK)], idx_v)

    def _zero(k, _):
        zeros_v[pl.ds(k * 16, 16)] = jnp.zeros((16,), jnp.float32)
        return _
    lax.fori_loop(0, (_NUM_EMBED // 16) // 16, _zero, 0)
    pltpu.sync_copy(zeros_v, cnt_sh.at[pl.ds(sid * (_NUM_EMBED // 16),
                                             _NUM_EMBED // 16)])

    for k in range(_CHUNK // 16):
        ones_v[pl.ds(k * 16, 16)] = jnp.ones((16,), jnp.float32)

    plsc.subcore_barrier()

    for j in range(_NCHUNK):
        pltpu.async_copy(ones_v, cnt_sh.at[idx_v.at[j]], sem, add=True).wait()

    for j in range(_NCHUNK):
        pltpu.async_copy(w_hbm.at[idx_v.at[j]], rows_v, sem).wait()
        pltpu.sync_copy(rows_v,
                        zq_hbm.at[pl.ds((wid * _NCHUNK + j) * _CHUNK, _CHUNK)])

    plsc.subcore_barrier()

    @pl.when(sid == 0)
    def _():
        pltpu.sync_copy(cnt_sh, cnt_hbm.at[cid])


def _epilogue_body(zq_ref, zp_ref, cnt_ref, zqst_ref, loss_ref, ppx_ref,
                   acc_ref):
    i = pl.program_id(0)
    zq = zq_ref[...]
    zp = zp_ref[...]
    diff = zq - zp
    zqst_ref[...] = zp + diff
    s = jnp.sum(diff * diff)

    @pl.when(i == 0)
    def _():
        acc_ref[0] = s

    @pl.when(i > 0)
    def _():
        acc_ref[0] = acc_ref[0] + s

    @pl.when(i == pl.num_programs(0) - 1)
    def _():
        m = acc_ref[0] / jnp.float32(_TOKENS * _EMBED_DIM)
        loss_ref[...] = jnp.reshape(jnp.float32(_BETA) * m + m, (1, 1))
        c = cnt_ref[0:1, :] + cnt_ref[1:2, :]
        p = c * jnp.float32(1.0 / _TOKENS)
        ent = jnp.sum(p * jnp.log(p + 1e-10))
        ppx_ref[...] = jnp.reshape(jnp.exp(-ent), (1, 1))


@functools.lru_cache(maxsize=1)
def _make_sc_gather_hist():
    return pl.kernel(
        _sc_body,
        out_type=(jax.ShapeDtypeStruct((_TOKENS, _EMBED_DIM), jnp.float32),
                  jax.ShapeDtypeStruct((2, _NUM_EMBED), jnp.float32)),
        mesh=plsc.VectorSubcoreMesh(core_axis_name="c", subcore_axis_name="s",
                                    num_cores=2, num_subcores=16),
        scratch_types=[
            pltpu.VMEM((_NCHUNK, _CHUNK), jnp.int32),
            pltpu.VMEM((_CHUNK, _EMBED_DIM), jnp.float32),
            pltpu.VMEM((_CHUNK,), jnp.float32),
            pltpu.VMEM((_NUM_EMBED // 16,), jnp.float32),
            pltpu.VMEM_SHARED((_NUM_EMBED,), jnp.float32),
            pltpu.SemaphoreType.DMA,
        ],
    )


def kernel(z, weight):
    b, c, h, w = z.shape
    zp = jnp.transpose(z, (0, 2, 3, 1)).reshape(-1, _EMBED_DIM)

    normed = pl.pallas_call(
        _normalize_body,
        out_shape=jax.ShapeDtypeStruct((_NUM_EMBED, _EMBED_DIM), jnp.float32),
    )(weight)

    n_tiles = _TOKENS // _TM
    idx3 = pl.pallas_call(
        _argmax_body,
        grid=(n_tiles,),
        in_specs=[
            pl.BlockSpec((_TM, _EMBED_DIM), lambda i: (i, 0)),
            pl.BlockSpec((_NUM_EMBED, _EMBED_DIM), lambda i: (0, 0)),
        ],
        out_specs=pl.BlockSpec((1, 1, _TM), lambda i: (i, 0, 0)),
        out_shape=jax.ShapeDtypeStruct((n_tiles, 1, _TM), jnp.int32),
    )(zp, normed)
    idx = idx3.reshape(_TOKENS)

    zq, cnt = _make_sc_gather_hist()(weight, idx.reshape(128, 128))

    n_epi = _TOKENS // _EPI_TM
    zqst, loss, ppx = pl.pallas_call(
        _epilogue_body,
        grid=(n_epi,),
        in_specs=[
            pl.BlockSpec((_EPI_TM, _EMBED_DIM), lambda i: (i, 0)),
            pl.BlockSpec((_EPI_TM, _EMBED_DIM), lambda i: (i, 0)),
            pl.BlockSpec((2, _NUM_EMBED), lambda i: (0, 0)),
        ],
        out_specs=[
            pl.BlockSpec((_EPI_TM, _EMBED_DIM), lambda i: (i, 0)),
            pl.BlockSpec((1, 1), lambda i: (0, 0)),
            pl.BlockSpec((1, 1), lambda i: (0, 0)),
        ],
        out_shape=[
            jax.ShapeDtypeStruct((_TOKENS, _EMBED_DIM), jnp.float32),
            jax.ShapeDtypeStruct((1, 1), jnp.float32),
            jax.ShapeDtypeStruct((1, 1), jnp.float32),
        ],
        scratch_shapes=[pltpu.SMEM((1,), jnp.float32)],
    )(zq, zp, cnt)

    z_q_out = zqst.reshape(b, h, w, c).transpose(0, 3, 1, 2)
    return z_q_out, loss[0, 0], ppx[0, 0], idx


# speedup vs baseline: 248.4658x; 1.3912x over previous
import functools

import jax
import jax.numpy as jnp
from jax import lax
from jax.experimental import pallas as pl
from jax.experimental.pallas import tpu as pltpu
from jax.experimental.pallas import tpu_sc as plsc

_NUM_EMBED = 8192
_EMBED_DIM = 256
_BETA = 0.25
_TOKENS = 16384

_TM = 256
_NW = 32
_TOK_PER_W = _TOKENS // _NW
_CHUNK = 128
_NCHUNK = _TOK_PER_W // _CHUNK
_EPI_TM = 2048


def _normalize_body(w_ref, out_ref):
    w = w_ref[...]
    n = jnp.sqrt(jnp.sum(w * w, axis=1, keepdims=True))
    out_ref[...] = w / jnp.maximum(n, 1e-12)


def _argmax_body(z_ref, nw_ref, idx_ref):
    zt = z_ref[...]
    n = jnp.sqrt(jnp.sum(zt * zt, axis=1, keepdims=True))
    zn = zt / jnp.maximum(n, 1e-12)
    scores = lax.dot_general(
        zn, nw_ref[...], (((1,), (1,)), ((), ())),
        preferred_element_type=jnp.float32)
    val = scores[:, 0:128]
    tid = jnp.zeros(val.shape, jnp.float32)
    for t in range(1, _NUM_EMBED // 128):
        tile = scores[:, t * 128:(t + 1) * 128]
        cond = tile >= val
        val = jnp.maximum(val, tile)
        tid = jnp.where(cond, jnp.float32(t), tid)
    m = jnp.max(val, axis=1, keepdims=True)
    lane = lax.broadcasted_iota(jnp.int32, val.shape, 1).astype(jnp.float32)
    g = tid * jnp.float32(128.0) + lane
    best = jnp.max(jnp.where(val == m, g, jnp.float32(-1.0)), axis=1)
    idx_ref[0, 0, :] = best.astype(jnp.int32)


def _sc_body(w_hbm, idx_hbm, zq_hbm, cnt_hbm,
             idx_v, rows_v, ones_v, zeros_v, cnt_sh, sem):
    cid = lax.axis_index("c")
    sid = lax.axis_index("s")
    wid = cid * 16 + sid

    pltpu.sync_copy(idx_hbm.at[pl.ds(wid * _NCHUNK, _NCHUNK)], idx_v)

    def _zero(k, _):
        zeros_v[pl.ds(k * 16, 16)] = jnp.zeros((16,), jnp.float32)
        return _
    lax.fori_loop(0, (_NUM_EMBED // 16) // 16, _zero, 0)
    pltpu.sync_copy(zeros_v, cnt_sh.at[pl.ds(sid * (_NUM_EMBED // 16),
                                             _NUM_EMBED // 16)])

    for k in range(_CHUNK // 16):
        ones_v[pl.ds(k * 16, 16)] = jnp.ones((16,), jnp.float32)

    plsc.subcore_barrier()

    for j in range(_NCHUNK):
        pltpu.async_copy(ones_v, cnt_sh.at[idx_v.at[j]], sem, add=True).wait()

    for j in range(_NCHUNK):
        pltpu.async_copy(w_hbm.at[idx_v.at[j]], rows_v, sem).wait()
        pltpu.sync_copy(rows_v,
                        zq_hbm.at[pl.ds((wid * _NCHUNK + j) * _CHUNK, _CHUNK)])

    plsc.subcore_barrier()

    @pl.when(sid == 0)
    def _():
        pltpu.sync_copy(cnt_sh, cnt_hbm.at[cid])


def _epilogue_body(zq_ref, zp_ref, cnt_ref, zqst_ref, loss_ref, ppx_ref,
                   acc_ref):
    i = pl.program_id(0)
    zq = zq_ref[...]
    zp = zp_ref[...]
    diff = zq - zp
    zqst_ref[...] = zp + diff
    s = jnp.sum(diff * diff)

    @pl.when(i == 0)
    def _():
        acc_ref[0] = s

    @pl.when(i > 0)
    def _():
        acc_ref[0] = acc_ref[0] + s

    @pl.when(i == pl.num_programs(0) - 1)
    def _():
        m = acc_ref[0] / jnp.float32(_TOKENS * _EMBED_DIM)
        loss_ref[...] = jnp.reshape(jnp.float32(_BETA) * m + m, (1, 1))
        c = cnt_ref[0:1, :] + cnt_ref[1:2, :]
        p = c * jnp.float32(1.0 / _TOKENS)
        ent = jnp.sum(p * jnp.log(p + 1e-10))
        ppx_ref[...] = jnp.reshape(jnp.exp(-ent), (1, 1))


@functools.lru_cache(maxsize=1)
def _make_sc_gather_hist():
    return pl.kernel(
        _sc_body,
        out_type=(jax.ShapeDtypeStruct((_TOKENS, _EMBED_DIM), jnp.float32),
                  jax.ShapeDtypeStruct((2, _NUM_EMBED), jnp.float32)),
        mesh=plsc.VectorSubcoreMesh(core_axis_name="c", subcore_axis_name="s",
                                    num_cores=2, num_subcores=16),
        scratch_types=[
            pltpu.VMEM((_NCHUNK, _CHUNK), jnp.int32),
            pltpu.VMEM((_CHUNK, _EMBED_DIM), jnp.float32),
            pltpu.VMEM((_CHUNK,), jnp.float32),
            pltpu.VMEM((_NUM_EMBED // 16,), jnp.float32),
            pltpu.VMEM_SHARED((_NUM_EMBED,), jnp.float32),
            pltpu.SemaphoreType.DMA,
        ],
    )


def kernel(z, weight):
    b, c, h, w = z.shape
    zp = jnp.transpose(z, (0, 2, 3, 1)).reshape(-1, _EMBED_DIM)

    normed = pl.pallas_call(
        _normalize_body,
        out_shape=jax.ShapeDtypeStruct((_NUM_EMBED, _EMBED_DIM), jnp.float32),
    )(weight)

    n_tiles = _TOKENS // _TM
    idx3 = pl.pallas_call(
        _argmax_body,
        grid=(n_tiles,),
        in_specs=[
            pl.BlockSpec((_TM, _EMBED_DIM), lambda i: (i, 0)),
            pl.BlockSpec((_NUM_EMBED, _EMBED_DIM), lambda i: (0, 0)),
        ],
        out_specs=pl.BlockSpec((1, 1, _TM), lambda i: (i, 0, 0)),
        out_shape=jax.ShapeDtypeStruct((n_tiles, 1, _TM), jnp.int32),
    )(zp, normed)
    idx = idx3.reshape(_TOKENS)

    zq, cnt = _make_sc_gather_hist()(weight, idx.reshape(128, 128))

    n_epi = _TOKENS // _EPI_TM
    zqst, loss, ppx = pl.pallas_call(
        _epilogue_body,
        grid=(n_epi,),
        in_specs=[
            pl.BlockSpec((_EPI_TM, _EMBED_DIM), lambda i: (i, 0)),
            pl.BlockSpec((_EPI_TM, _EMBED_DIM), lambda i: (i, 0)),
            pl.BlockSpec((2, _NUM_EMBED), lambda i: (0, 0)),
        ],
        out_specs=[
            pl.BlockSpec((_EPI_TM, _EMBED_DIM), lambda i: (i, 0)),
            pl.BlockSpec((1, 1), lambda i: (0, 0)),
            pl.BlockSpec((1, 1), lambda i: (0, 0)),
        ],
        out_shape=[
            jax.ShapeDtypeStruct((_TOKENS, _EMBED_DIM), jnp.float32),
            jax.ShapeDtypeStruct((1, 1), jnp.float32),
            jax.ShapeDtypeStruct((1, 1), jnp.float32),
        ],
        scratch_shapes=[pltpu.SMEM((1,), jnp.float32)],
    )(zq, zp, cnt)

    z_q_out = zqst.reshape(b, h, w, c).transpose(0, 3, 1, 2)
    return z_q_out, loss[0, 0], ppx[0, 0], idx
